# baseline (speedup 1.0000x reference)
import functools

import jax
import jax.numpy as jnp
from jax import lax
from jax.experimental import pallas as pl
from jax.experimental.pallas import tpu as pltpu
from jax.experimental.pallas import tpu_sc as plsc

N = 10000
E = 320000
D = 128
G = 64
ED = 16

NC = 2
NS = 16
NW = NC * NS
EPW = E // NW
BLK_E = 80
NBLK = EPW // BLK_E
ROWS_PER_TILE = 624
TAIL_ROWS = N - NS * ROWS_PER_TILE
TAIL_OFF = NS * ROWS_PER_TILE


def _edge_lin_body(ea_ref, w1_ref, b1_ref, w2_ref, b2_ref, e1_ref, e2_ref):
    a = ea_ref[...]
    e1_ref[...] = jnp.dot(a, w1_ref[...], preferred_element_type=jnp.float32) + b1_ref[...]
    e2_ref[...] = jnp.dot(a, w2_ref[...], preferred_element_type=jnp.float32) + b2_ref[...]


def _edge_lin(ea, w1, b1, w2, b2):
    BLK = 4000
    grid = E // BLK
    return pl.pallas_call(
        _edge_lin_body,
        grid=(grid,),
        in_specs=[
            pl.BlockSpec((BLK, ED), lambda i: (i, 0)),
            pl.BlockSpec((ED, D), lambda i: (0, 0)),
            pl.BlockSpec((1, D), lambda i: (0, 0)),
            pl.BlockSpec((ED, D), lambda i: (0, 0)),
            pl.BlockSpec((1, D), lambda i: (0, 0)),
        ],
        out_specs=[
            pl.BlockSpec((BLK, D), lambda i: (i, 0)),
            pl.BlockSpec((BLK, D), lambda i: (i, 0)),
        ],
        out_shape=[jax.ShapeDtypeStruct((E, D), jnp.float32)] * 2,
    )(ea, w1, b1.reshape(1, D), w2, b2.reshape(1, D))


def _sc_agg_body(table, src, dst, e, zeros, out,
                 agg_sh, src_v, dst_v, rows_v, e_v,
                 sem_i, sem_g, sem_e, sem_s):
    c = lax.axis_index("c")
    s = lax.axis_index("s")
    wid = s * NC + c

    pltpu.sync_copy(zeros.at[pl.ds(s * ROWS_PER_TILE, ROWS_PER_TILE)],
                    agg_sh.at[pl.ds(s * ROWS_PER_TILE, ROWS_PER_TILE)])

    @pl.when(s == 0)
    def _():
        pltpu.sync_copy(zeros.at[pl.ds(TAIL_OFF, TAIL_ROWS)],
                        agg_sh.at[pl.ds(TAIL_OFF, TAIL_ROWS)])

    plsc.subcore_barrier()

    def _base(i):
        return pl.multiple_of(wid * EPW + i * BLK_E, 8)

    def _idx_start(i, slot):
        b = _base(i)
        pltpu.make_async_copy(src.at[pl.ds(b, BLK_E)], src_v.at[slot],
                              sem_i.at[slot]).start()
        pltpu.make_async_copy(dst.at[pl.ds(b, BLK_E)], dst_v.at[slot],
                              sem_i.at[slot]).start()

    def _idx_wait(i, slot):
        b = _base(i)
        pltpu.make_async_copy(src.at[pl.ds(b, BLK_E)], src_v.at[slot],
                              sem_i.at[slot]).wait()
        pltpu.make_async_copy(dst.at[pl.ds(b, BLK_E)], dst_v.at[slot],
                              sem_i.at[slot]).wait()

    def _data_start(i, islot, b):
        pltpu.make_async_copy(table.at[src_v.at[islot]], rows_v.at[b],
                              sem_g.at[b]).start()
        pltpu.make_async_copy(e.at[pl.ds(_base(i), BLK_E)], e_v.at[b],
                              sem_e.at[b]).start()

    def _scatter_drain(b):
        pltpu.make_async_copy(zeros.at[pl.ds(0, BLK_E)], rows_v.at[b],
                              sem_s.at[b]).wait()

    pltpu.sync_copy(src.at[pl.ds(_base(0), BLK_E)], src_v.at[0])
    pltpu.sync_copy(dst.at[pl.ds(_base(0), BLK_E)], dst_v.at[0])
    _idx_start(1, 1)
    _data_start(0, 0, 0)

    def blk(i, carry):
        b = i % 2
        nb = (i + 1) % 2

        @pl.when(i < NBLK - 1)
        def _():
            _idx_wait(i + 1, (i + 1) % 4)

            @pl.when(i >= 1)
            def _():
                _scatter_drain(nb)

            _data_start(i + 1, (i + 1) % 4, nb)

        @pl.when(i < NBLK - 2)
        def _():
            _idx_start(i + 2, (i + 2) % 4)

        pltpu.make_async_copy(table.at[src_v.at[i % 4]], rows_v.at[b],
                              sem_g.at[b]).wait()
        pltpu.make_async_copy(e.at[pl.ds(_base(i), BLK_E)], e_v.at[b],
                              sem_e.at[b]).wait()

        @plsc.parallel_loop(0, BLK_E, 1, unroll=6)
        def _edge(k):
            for j in range(D // 16):
                sl = pl.ds(j * 16, 16)
                v = rows_v[b, k, sl] + e_v[b, k, sl]
                rows_v[b, k, sl] = jnp.maximum(v, 0.0)

        pltpu.async_copy(rows_v.at[b], agg_sh.at[dst_v.at[i % 4]],
                         sem_s.at[b], add=True)
        return carry

    lax.fori_loop(0, NBLK, blk, 0)
    _scatter_drain(0)
    _scatter_drain(1)
    plsc.subcore_barrier()
    pltpu.sync_copy(agg_sh.at[pl.ds(s * ROWS_PER_TILE, ROWS_PER_TILE)],
                    out.at[c, pl.ds(s * ROWS_PER_TILE, ROWS_PER_TILE)])

    @pl.when(s == 0)
    def _():
        pltpu.sync_copy(agg_sh.at[pl.ds(TAIL_OFF, TAIL_ROWS)],
                        out.at[c, pl.ds(TAIL_OFF, TAIL_ROWS)])


_sc_agg = functools.partial(
    pl.kernel,
    out_type=jax.ShapeDtypeStruct((NC, N, D), jnp.float32),
    mesh=plsc.VectorSubcoreMesh(core_axis_name="c", subcore_axis_name="s"),
    scratch_types=[
        pltpu.VMEM_SHARED((N, D), jnp.float32),
        pltpu.VMEM((4, BLK_E), jnp.int32),
        pltpu.VMEM((4, BLK_E), jnp.int32),
        pltpu.VMEM((2, BLK_E, D), jnp.float32),
        pltpu.VMEM((2, BLK_E, D), jnp.float32),
        pltpu.SemaphoreType.DMA((4,)),
        pltpu.SemaphoreType.DMA((2,)),
        pltpu.SemaphoreType.DMA((2,)),
        pltpu.SemaphoreType.DMA((2,)),
    ],
)(_sc_agg_body)


BLK_N = 1000
GRID_N = N // BLK_N


def _mlp_body(x_ref, a0_ref, a1_ref, w1_ref, b1_ref, w2_ref, b2_ref, o_ref):
    t = x_ref[...] + a0_ref[...] + a1_ref[...]
    t = jnp.maximum(jnp.dot(t, w1_ref[...], preferred_element_type=jnp.float32) + b1_ref[...], 0.0)
    t = jnp.dot(t, w2_ref[...], preferred_element_type=jnp.float32) + b2_ref[...]
    o_ref[...] = jnp.maximum(t, 0.0)


def _mlp(x, a0, a1, w1, b1, w2, b2):
    return pl.pallas_call(
        _mlp_body,
        grid=(GRID_N,),
        in_specs=[
            pl.BlockSpec((BLK_N, D), lambda i: (i, 0)),
            pl.BlockSpec((BLK_N, D), lambda i: (i, 0)),
            pl.BlockSpec((BLK_N, D), lambda i: (i, 0)),
            pl.BlockSpec((D, D), lambda i: (0, 0)),
            pl.BlockSpec((1, D), lambda i: (0, 0)),
            pl.BlockSpec((D, D), lambda i: (0, 0)),
            pl.BlockSpec((1, D), lambda i: (0, 0)),
        ],
        out_specs=pl.BlockSpec((BLK_N, D), lambda i: (i, 0)),
        out_shape=jax.ShapeDtypeStruct((N, D), jnp.float32),
    )(x, a0, a1, w1, b1.reshape(1, D), w2, b2.reshape(1, D))


def _mlp_pool_body(x_ref, a0_ref, a1_ref, w1_ref, b1_ref, w2_ref, b2_ref,
                   batch_ref, fcw_ref, fcb_ref, o_ref, acc_ref, cnt_ref):
    i = pl.program_id(0)

    @pl.when(i == 0)
    def _():
        acc_ref[...] = jnp.zeros_like(acc_ref)
        cnt_ref[...] = jnp.zeros_like(cnt_ref)

    t = x_ref[...] + a0_ref[...] + a1_ref[...]
    t = jnp.maximum(jnp.dot(t, w1_ref[...], preferred_element_type=jnp.float32) + b1_ref[...], 0.0)
    t = jnp.dot(t, w2_ref[...], preferred_element_type=jnp.float32) + b2_ref[...]
    t = jnp.maximum(t, 0.0)

    b = batch_ref[0, 0, :]
    gids = lax.broadcasted_iota(jnp.int32, (G, BLK_N), 0)
    onehot = (b[None, :] == gids).astype(jnp.float32)
    acc_ref[...] += jnp.dot(onehot, t, preferred_element_type=jnp.float32)
    cnt_ref[...] += jnp.sum(onehot, axis=1, keepdims=True)

    @pl.when(i == GRID_N - 1)
    def _():
        mean = acc_ref[...] / jnp.maximum(cnt_ref[...], 1.0)
        o_ref[...] = jnp.dot(mean, fcw_ref[...], preferred_element_type=jnp.float32) + fcb_ref[...]


def _mlp_pool(x, a0, a1, w1, b1, w2, b2, batch, fcw, fcb):
    return pl.pallas_call(
        _mlp_pool_body,
        grid=(GRID_N,),
        in_specs=[
            pl.BlockSpec((BLK_N, D), lambda i: (i, 0)),
            pl.BlockSpec((BLK_N, D), lambda i: (i, 0)),
            pl.BlockSpec((BLK_N, D), lambda i: (i, 0)),
            pl.BlockSpec((D, D), lambda i: (0, 0)),
            pl.BlockSpec((1, D), lambda i: (0, 0)),
            pl.BlockSpec((D, D), lambda i: (0, 0)),
            pl.BlockSpec((1, D), lambda i: (0, 0)),
            pl.BlockSpec((1, 1, BLK_N), lambda i: (i, 0, 0)),
            pl.BlockSpec((D, D), lambda i: (0, 0)),
            pl.BlockSpec((1, D), lambda i: (0, 0)),
        ],
        out_specs=pl.BlockSpec((G, D), lambda i: (0, 0)),
        out_shape=jax.ShapeDtypeStruct((G, D), jnp.float32),
        scratch_shapes=[
            pltpu.VMEM((G, D), jnp.float32),
            pltpu.VMEM((G, 1), jnp.float32),
        ],
    )(x, a0, a1, w1, b1.reshape(1, D), w2, b2.reshape(1, D),
      batch.reshape(GRID_N, 1, BLK_N), fcw, fcb.reshape(1, D))


def kernel(x, edge_index, edge_attr, batch,
           le1_W, le1_b, m1_W1, m1_b1, m1_W2, m1_b2,
           le2_W, le2_b, m2_W1, m2_b1, m2_W2, m2_b2,
           fc_W, fc_b):
    src = edge_index[0]
    dst = edge_index[1]
    e1, e2 = _edge_lin(edge_attr, le1_W, le1_b, le2_W, le2_b)
    zeros = jnp.zeros((N, D), jnp.float32)

    agg1 = _sc_agg(x, src, dst, e1, zeros)
    h = _mlp(x, agg1[0], agg1[1], m1_W1, m1_b1, m1_W2, m1_b2)

    agg2 = _sc_agg(h, src, dst, e2, zeros)
    out = _mlp_pool(h, agg2[0], agg2[1], m2_W1, m2_b1, m2_W2, m2_b2,
                    batch, fc_W, fc_b)
    return out


# speedup vs baseline: 1.0327x; 1.0327x over previous
import functools

import jax
import jax.numpy as jnp
from jax import lax
from jax.experimental import pallas as pl
from jax.experimental.pallas import tpu as pltpu
from jax.experimental.pallas import tpu_sc as plsc

N = 10000
E = 320000
D = 128
G = 64
ED = 16

NC = 2
NS = 16
NW = NC * NS
EPW = E // NW
BLK_E = 80
NBLK = EPW // BLK_E
ROWS_PER_TILE = 624
TAIL_ROWS = N - NS * ROWS_PER_TILE
TAIL_OFF = NS * ROWS_PER_TILE


def _edge_lin_body(ea_ref, w1_ref, b1_ref, w2_ref, b2_ref, e1_ref, e2_ref):
    a = ea_ref[...]
    e1_ref[...] = jnp.dot(a, w1_ref[...], preferred_element_type=jnp.float32) + b1_ref[...]
    e2_ref[...] = jnp.dot(a, w2_ref[...], preferred_element_type=jnp.float32) + b2_ref[...]


def _edge_lin(ea, w1, b1, w2, b2):
    BLK = 4000
    grid = E // BLK
    return pl.pallas_call(
        _edge_lin_body,
        grid=(grid,),
        in_specs=[
            pl.BlockSpec((BLK, ED), lambda i: (i, 0)),
            pl.BlockSpec((ED, D), lambda i: (0, 0)),
            pl.BlockSpec((1, D), lambda i: (0, 0)),
            pl.BlockSpec((ED, D), lambda i: (0, 0)),
            pl.BlockSpec((1, D), lambda i: (0, 0)),
        ],
        out_specs=[
            pl.BlockSpec((BLK, D), lambda i: (i, 0)),
            pl.BlockSpec((BLK, D), lambda i: (i, 0)),
        ],
        out_shape=[jax.ShapeDtypeStruct((E, D), jnp.float32)] * 2,
    )(ea, w1, b1.reshape(1, D), w2, b2.reshape(1, D))


def _sc_agg_body(table, src, dst, e, zeros, out,
                 agg_sh, src_v, dst_v, rows_v, e_v,
                 sem_i, sem_g, sem_e, sem_s):
    c = lax.axis_index("c")
    s = lax.axis_index("s")
    wid = s * NC + c

    pltpu.sync_copy(zeros.at[pl.ds(s * ROWS_PER_TILE, ROWS_PER_TILE)],
                    agg_sh.at[pl.ds(s * ROWS_PER_TILE, ROWS_PER_TILE)])

    @pl.when(s == 0)
    def _():
        pltpu.sync_copy(zeros.at[pl.ds(TAIL_OFF, TAIL_ROWS)],
                        agg_sh.at[pl.ds(TAIL_OFF, TAIL_ROWS)])

    plsc.subcore_barrier()

    def _base(i):
        return pl.multiple_of(wid * EPW + i * BLK_E, 8)

    def _idx_start(i, slot):
        b = _base(i)
        pltpu.make_async_copy(src.at[pl.ds(b, BLK_E)], src_v.at[slot],
                              sem_i.at[slot]).start()
        pltpu.make_async_copy(dst.at[pl.ds(b, BLK_E)], dst_v.at[slot],
                              sem_i.at[slot]).start()

    def _idx_wait(i, slot):
        b = _base(i)
        pltpu.make_async_copy(src.at[pl.ds(b, BLK_E)], src_v.at[slot],
                              sem_i.at[slot]).wait()
        pltpu.make_async_copy(dst.at[pl.ds(b, BLK_E)], dst_v.at[slot],
                              sem_i.at[slot]).wait()

    def _data_start(i, islot, b):
        pltpu.make_async_copy(table.at[src_v.at[islot]], rows_v.at[b],
                              sem_g.at[b]).start()
        pltpu.make_async_copy(e.at[pl.ds(_base(i), BLK_E)], e_v.at[b],
                              sem_e.at[b]).start()

    def _scatter_drain(b):
        pltpu.make_async_copy(zeros.at[pl.ds(0, BLK_E)], rows_v.at[b],
                              sem_s.at[b]).wait()

    pltpu.sync_copy(src.at[pl.ds(_base(0), BLK_E)], src_v.at[0])
    pltpu.sync_copy(dst.at[pl.ds(_base(0), BLK_E)], dst_v.at[0])
    _idx_start(1, 1)
    _data_start(0, 0, 0)

    def blk(i, carry):
        b = i % 2
        nb = (i + 1) % 2

        @pl.when(i < NBLK - 1)
        def _():
            _idx_wait(i + 1, (i + 1) % 4)

            @pl.when(i >= 1)
            def _():
                _scatter_drain(nb)

            _data_start(i + 1, (i + 1) % 4, nb)

        @pl.when(i < NBLK - 2)
        def _():
            _idx_start(i + 2, (i + 2) % 4)

        pltpu.make_async_copy(table.at[src_v.at[i % 4]], rows_v.at[b],
                              sem_g.at[b]).wait()
        pltpu.make_async_copy(e.at[pl.ds(_base(i), BLK_E)], e_v.at[b],
                              sem_e.at[b]).wait()

        @plsc.parallel_loop(0, BLK_E, 1, unroll=4)
        def _edge(k):
            for j in range(D // 16):
                sl = pl.ds(j * 16, 16)
                v = rows_v[b, k, sl] + e_v[b, k, sl]
                rows_v[b, k, sl] = jnp.maximum(v, 0.0)

        pltpu.async_copy(rows_v.at[b], agg_sh.at[dst_v.at[i % 4]],
                         sem_s.at[b], add=True)
        return carry

    lax.fori_loop(0, NBLK, blk, 0)
    _scatter_drain(0)
    _scatter_drain(1)
    plsc.subcore_barrier()
    pltpu.sync_copy(agg_sh.at[pl.ds(s * ROWS_PER_TILE, ROWS_PER_TILE)],
                    out.at[c, pl.ds(s * ROWS_PER_TILE, ROWS_PER_TILE)])

    @pl.when(s == 0)
    def _():
        pltpu.sync_copy(agg_sh.at[pl.ds(TAIL_OFF, TAIL_ROWS)],
                        out.at[c, pl.ds(TAIL_OFF, TAIL_ROWS)])


_sc_agg = functools.partial(
    pl.kernel,
    out_type=jax.ShapeDtypeStruct((NC, N, D), jnp.float32),
    mesh=plsc.VectorSubcoreMesh(core_axis_name="c", subcore_axis_name="s"),
    scratch_types=[
        pltpu.VMEM_SHARED((N, D), jnp.float32),
        pltpu.VMEM((4, BLK_E), jnp.int32),
        pltpu.VMEM((4, BLK_E), jnp.int32),
        pltpu.VMEM((2, BLK_E, D), jnp.float32),
        pltpu.VMEM((2, BLK_E, D), jnp.float32),
        pltpu.SemaphoreType.DMA((4,)),
        pltpu.SemaphoreType.DMA((2,)),
        pltpu.SemaphoreType.DMA((2,)),
        pltpu.SemaphoreType.DMA((2,)),
    ],
)(_sc_agg_body)


BLK_N = 1000
GRID_N = N // BLK_N


def _mlp_body(x_ref, a0_ref, a1_ref, w1_ref, b1_ref, w2_ref, b2_ref, o_ref):
    t = x_ref[...] + a0_ref[...] + a1_ref[...]
    t = jnp.maximum(jnp.dot(t, w1_ref[...], preferred_element_type=jnp.float32) + b1_ref[...], 0.0)
    t = jnp.dot(t, w2_ref[...], preferred_element_type=jnp.float32) + b2_ref[...]
    o_ref[...] = jnp.maximum(t, 0.0)


def _mlp(x, a0, a1, w1, b1, w2, b2):
    return pl.pallas_call(
        _mlp_body,
        grid=(GRID_N,),
        in_specs=[
            pl.BlockSpec((BLK_N, D), lambda i: (i, 0)),
            pl.BlockSpec((BLK_N, D), lambda i: (i, 0)),
            pl.BlockSpec((BLK_N, D), lambda i: (i, 0)),
            pl.BlockSpec((D, D), lambda i: (0, 0)),
            pl.BlockSpec((1, D), lambda i: (0, 0)),
            pl.BlockSpec((D, D), lambda i: (0, 0)),
            pl.BlockSpec((1, D), lambda i: (0, 0)),
        ],
        out_specs=pl.BlockSpec((BLK_N, D), lambda i: (i, 0)),
        out_shape=jax.ShapeDtypeStruct((N, D), jnp.float32),
    )(x, a0, a1, w1, b1.reshape(1, D), w2, b2.reshape(1, D))


def _mlp_pool_body(x_ref, a0_ref, a1_ref, w1_ref, b1_ref, w2_ref, b2_ref,
                   batch_ref, fcw_ref, fcb_ref, o_ref, acc_ref, cnt_ref):
    i = pl.program_id(0)

    @pl.when(i == 0)
    def _():
        acc_ref[...] = jnp.zeros_like(acc_ref)
        cnt_ref[...] = jnp.zeros_like(cnt_ref)

    t = x_ref[...] + a0_ref[...] + a1_ref[...]
    t = jnp.maximum(jnp.dot(t, w1_ref[...], preferred_element_type=jnp.float32) + b1_ref[...], 0.0)
    t = jnp.dot(t, w2_ref[...], preferred_element_type=jnp.float32) + b2_ref[...]
    t = jnp.maximum(t, 0.0)

    b = batch_ref[0, 0, :]
    gids = lax.broadcasted_iota(jnp.int32, (G, BLK_N), 0)
    onehot = (b[None, :] == gids).astype(jnp.float32)
    acc_ref[...] += jnp.dot(onehot, t, preferred_element_type=jnp.float32)
    cnt_ref[...] += jnp.sum(onehot, axis=1, keepdims=True)

    @pl.when(i == GRID_N - 1)
    def _():
        mean = acc_ref[...] / jnp.maximum(cnt_ref[...], 1.0)
        o_ref[...] = jnp.dot(mean, fcw_ref[...], preferred_element_type=jnp.float32) + fcb_ref[...]


def _mlp_pool(x, a0, a1, w1, b1, w2, b2, batch, fcw, fcb):
    return pl.pallas_call(
        _mlp_pool_body,
        grid=(GRID_N,),
        in_specs=[
            pl.BlockSpec((BLK_N, D), lambda i: (i, 0)),
            pl.BlockSpec((BLK_N, D), lambda i: (i, 0)),
            pl.BlockSpec((BLK_N, D), lambda i: (i, 0)),
            pl.BlockSpec((D, D), lambda i: (0, 0)),
            pl.BlockSpec((1, D), lambda i: (0, 0)),
            pl.BlockSpec((D, D), lambda i: (0, 0)),
            pl.BlockSpec((1, D), lambda i: (0, 0)),
            pl.BlockSpec((1, 1, BLK_N), lambda i: (i, 0, 0)),
            pl.BlockSpec((D, D), lambda i: (0, 0)),
            pl.BlockSpec((1, D), lambda i: (0, 0)),
        ],
        out_specs=pl.BlockSpec((G, D), lambda i: (0, 0)),
        out_shape=jax.ShapeDtypeStruct((G, D), jnp.float32),
        scratch_shapes=[
            pltpu.VMEM((G, D), jnp.float32),
            pltpu.VMEM((G, 1), jnp.float32),
        ],
    )(x, a0, a1, w1, b1.reshape(1, D), w2, b2.reshape(1, D),
      batch.reshape(GRID_N, 1, BLK_N), fcw, fcb.reshape(1, D))


def kernel(x, edge_index, edge_attr, batch,
           le1_W, le1_b, m1_W1, m1_b1, m1_W2, m1_b2,
           le2_W, le2_b, m2_W1, m2_b1, m2_W2, m2_b2,
           fc_W, fc_b):
    src = edge_index[0]
    dst = edge_index[1]
    e1, e2 = _edge_lin(edge_attr, le1_W, le1_b, le2_W, le2_b)
    zeros = jnp.zeros((N, D), jnp.float32)

    agg1 = _sc_agg(x, src, dst, e1, zeros)
    h = _mlp(x, agg1[0], agg1[1], m1_W1, m1_b1, m1_W2, m1_b2)

    agg2 = _sc_agg(h, src, dst, e2, zeros)
    out = _mlp_pool(h, agg2[0], agg2[1], m2_W1, m2_b1, m2_W2, m2_b2,
                    batch, fc_W, fc_b)
    return out
